# Initial kernel scaffold; baseline (speedup 1.0000x reference)
import jax
import jax.numpy as jnp
from jax.experimental import pallas as pl


def kernel(x, edge_index, W1, a_src1, a_dst1, b1, W2, a_src2, a_dst2, b2):
    raise NotImplementedError("write your pallas kernel here")


# speedup vs baseline: 1.1661x; 1.1661x over previous
import functools

import jax
import jax.numpy as jnp
from jax.experimental import pallas as pl

N_NODES = 10000
N_PAD = 10240
IN_DIM = 128
HID_DIM = 64
OUT_DIM = 3
HEADS = 8


def _mm_kernel(x_ref, w_ref, o_ref):
    o_ref[...] = jnp.dot(x_ref[...], w_ref[...],
                         preferred_element_type=jnp.float32)


def _matmul(x, w):
    m, k = x.shape
    n = w.shape[1]
    blk = 640
    return pl.pallas_call(
        _mm_kernel,
        grid=(m // blk,),
        in_specs=[
            pl.BlockSpec((blk, k), lambda i: (i, 0)),
            pl.BlockSpec((k, n), lambda i: (0, 0)),
        ],
        out_specs=pl.BlockSpec((blk, n), lambda i: (i, 0)),
        out_shape=jax.ShapeDtypeStruct((m, n), jnp.float32),
    )(x, w)


def _gat_layer(x, src, dst, W, a_src, a_dst, bias, heads, out_ch):
    n = x.shape[0]
    h = _matmul(x, W).reshape(n, heads, out_ch)
    alpha_src = jnp.sum(h * a_src, axis=-1)
    alpha_dst = jnp.sum(h * a_dst, axis=-1)
    e = alpha_src[src] + alpha_dst[dst]
    e = jnp.maximum(e, 0.2 * e)
    w_e = jnp.exp(e)
    denom = jax.ops.segment_sum(w_e, dst, num_segments=n)
    msg = h[src] * w_e[:, :, None]
    out = jax.ops.segment_sum(msg, dst, num_segments=n)
    out = out / (denom[:, :, None] + 1e-16)
    return out, bias


def kernel(x, edge_index, W1, a_src1, a_dst1, b1, W2, a_src2, a_dst2, b2):
    n = x.shape[0]
    loop = jnp.arange(n, dtype=edge_index.dtype)
    src = jnp.concatenate([edge_index[0], loop])
    dst = jnp.concatenate([edge_index[1], loop])
    xp = jnp.pad(x, ((0, N_PAD - n), (0, 0)))

    out1, _ = _gat_layer(xp, src, dst, W1, a_src1, a_dst1, b1, HEADS, HID_DIM)
    h1 = jax.nn.elu(out1.reshape(N_PAD, HEADS * HID_DIM) + b1)

    out2, _ = _gat_layer(h1, src, dst, W2, a_src2, a_dst2, b2, 1, OUT_DIM)
    logits = out2.reshape(N_PAD, OUT_DIM)[:n] + b2
    return jax.nn.log_softmax(logits, axis=1)


# speedup vs baseline: 12.2673x; 10.5199x over previous
import functools

import jax
import jax.numpy as jnp
from jax import lax
from jax.experimental import pallas as pl
from jax.experimental.pallas import tpu as pltpu
from jax.experimental.pallas import tpu_sc as plsc

N = 10000
NP = 10240
E = 320000
EP = 344064
EROWS = EP // 128
RPT = EROWS // 16
HEADS = 8
HID = 64
ODIM = 3
NB = 640

_SCPARAMS = pltpu.CompilerParams(needs_layout_passes=False,
                                 use_tc_tiling_on_sc=False)
_MESH = plsc.VectorSubcoreMesh(core_axis_name="c", subcore_axis_name="s")


def _stage_a_body(x_ref, w_ref, asr_ref, adr_ref, hh_ref, aso_ref, ado_ref):
    h = jnp.dot(x_ref[...], w_ref[...], preferred_element_type=jnp.float32)
    for hd in range(HEADS):
        hh = h[:, hd * HID:(hd + 1) * HID]
        hh_ref[hd] = hh
        aso_ref[hd, :] = jnp.sum(hh * asr_ref[0, hd, :][None, :], axis=1)
        ado_ref[hd, :] = jnp.sum(hh * adr_ref[0, hd, :][None, :], axis=1)


def _stage_a(xp, W1, a_src1, a_dst1):
    return pl.pallas_call(
        _stage_a_body,
        grid=(NP // NB,),
        in_specs=[
            pl.BlockSpec((NB, 128), lambda i: (i, 0)),
            pl.BlockSpec((128, HEADS * HID), lambda i: (0, 0)),
            pl.BlockSpec((1, HEADS, HID), lambda i: (0, 0, 0)),
            pl.BlockSpec((1, HEADS, HID), lambda i: (0, 0, 0)),
        ],
        out_specs=[
            pl.BlockSpec((HEADS, NB, HID), lambda i: (0, i, 0)),
            pl.BlockSpec((HEADS, NB), lambda i: (0, i)),
            pl.BlockSpec((HEADS, NB), lambda i: (0, i)),
        ],
        out_shape=[
            jax.ShapeDtypeStruct((HEADS, NP, HID), jnp.float32),
            jax.ShapeDtypeStruct((HEADS, NP), jnp.float32),
            jax.ShapeDtypeStruct((HEADS, NP), jnp.float32),
        ],
    )(xp, W1, a_src1, a_dst1)


_GROUPS_W = RPT // 8


@functools.partial(
    pl.kernel,
    out_type=[
        jax.ShapeDtypeStruct((HEADS, EROWS, 128), jnp.float32),
        jax.ShapeDtypeStruct((HEADS, NP), jnp.float32),
    ],
    mesh=_MESH,
    scratch_types=[
        pltpu.VMEM((80, 128), jnp.float32),
        pltpu.VMEM((80, 128), jnp.float32),
        pltpu.VMEM((8, 128), jnp.int32),
        pltpu.VMEM((8, 128), jnp.int32),
        pltpu.VMEM((8, 128), jnp.float32),
        pltpu.VMEM((1024, 16), jnp.float32),
        pltpu.VMEM((640, 16), jnp.float32),
        pltpu.VMEM((640,), jnp.float32),
        pltpu.VMEM((16,), jnp.int32),
        pltpu.VMEM((80,), jnp.int32),
        pltpu.VMEM_SHARED((NP, 16), jnp.float32),
        pltpu.SemaphoreType.DMA,
        pltpu.SemaphoreType.DMA,
    ],
    compiler_params=_SCPARAMS,
)
def _w_kernel(aso2, ado2, pk2d, wout, den,
              at1, at2, srci, dsti, wrow, rden, dbuf, den1, ibuf, ib80,
              accd, isem, ssem):
    iota16 = jnp.arange(16, dtype=jnp.int32)
    c = lax.axis_index("c")
    s = lax.axis_index("s")

    def zb(i, _):
        rden[i, pl.ds(0, 16)] = jnp.zeros((16,), jnp.float32)
        return 0
    lax.fori_loop(0, 128, zb, 0)
    for z in range(5):
        pltpu.sync_copy(rden.at[pl.ds(0, 128)],
                        accd.at[pl.ds(s * NB + z * 128, 128)])
    plsc.subcore_barrier()

    def headw(hh, _hw):
        hd = c * 4 + hh
        base80 = hd * 80
        for z in range(5):
            ib80[pl.ds(z * 16, 16)] = base80 + z * 16 + iota16
        pltpu.async_copy(aso2.at[ib80], at1, isem).wait()
        pltpu.async_copy(ado2.at[ib80], at2, isem).wait()

        def group(g, _):
            row0 = s * RPT + g * 8
            ibuf[pl.ds(0, 16)] = row0 + iota16
            pltpu.async_copy(pk2d.at[ibuf.at[pl.ds(0, 8)]], srci, isem).wait()

            def unp(i2, _):
                for j in range(8):
                    v = srci[j, pl.ds(i2 * 16, 16)]
                    dsti[j, pl.ds(i2 * 16, 16)] = v >> 14
                    srci[j, pl.ds(i2 * 16, 16)] = v & 16383
                return 0
            lax.fori_loop(0, 8, unp, 0)

            for j in range(8):
                def wb(i2, _):
                    si = srci[j, pl.ds(i2 * 16, 16)]
                    di = dsti[j, pl.ds(i2 * 16, 16)]
                    a = plsc.load_gather(at1, [si >> 7, si & 127])
                    b = plsc.load_gather(at2, [di >> 7, di & 127])
                    e = a + b
                    e = jnp.maximum(e, e * 0.2)
                    wv = jnp.exp(e)
                    wrow[j, pl.ds(i2 * 16, 16)] = wv

                    for k in range(16):
                        e2 = j * 128 + i2 * 16 + k
                        rden[e2, pl.ds(0, 16)] = jnp.where(
                            iota16 == hh, wv[k], 0.0)
                    return 0
                lax.fori_loop(0, 8, wb, 0)

            pltpu.sync_copy(wrow, wout.at[hd, pl.ds(row0, 8)])
            scps = [
                pltpu.async_copy(rden.at[pl.ds(j * 128, 128)],
                                 accd.at[dsti.at[j]], ssem, add=True)
                for j in range(8)
            ]
            for cp in scps:
                cp.wait()
            return 0
        lax.fori_loop(0, _GROUPS_W, group, 0)
        return 0
    lax.fori_loop(0, 4, headw, 0)

    plsc.subcore_barrier()
    pltpu.sync_copy(accd.at[pl.ds(s * NB, NB)], dbuf)
    def headx(hh, _hx):
        hd = c * 4 + hh
        hv = jnp.zeros((16,), jnp.int32) + hh

        def dx(i, _):
            iv = i * 16 + iota16
            den1[pl.ds(i * 16, 16)] = plsc.load_gather(dbuf, [iv, hv])
            return 0
        lax.fori_loop(0, 40, dx, 0)
        pltpu.sync_copy(den1, den.at[hd, pl.ds(s * NB, NB)])
        return 0
    lax.fori_loop(0, 4, headx, 0)


_GROUPS_M = RPT // 2


@functools.partial(
    pl.kernel,
    out_type=jax.ShapeDtypeStruct((HEADS, NP, HID), jnp.float32),
    mesh=_MESH,
    scratch_types=[
        pltpu.VMEM((256, HID), jnp.float32),
        pltpu.VMEM((2, 128), jnp.int32),
        pltpu.VMEM((2, 128), jnp.int32),
        pltpu.VMEM((2, 128), jnp.float32),
        pltpu.VMEM((5, 128), jnp.float32),
        pltpu.VMEM((8, 128), jnp.float32),
        pltpu.VMEM((16,), jnp.int32),
        pltpu.VMEM_SHARED((NP, HID), jnp.float32),
        pltpu.SemaphoreType.DMA,
        pltpu.SemaphoreType.DMA,
        pltpu.SemaphoreType.DMA,
    ],
    compiler_params=_SCPARAMS,
)
def _m_kernel(hrows, pk2d, w2d, den2d, b1in, out1,
              rows, srci, dsti, wbuf, dbuf, bbuf, ibuf,
              acc, gsem, ssem, isem):
    iota16 = jnp.arange(16, dtype=jnp.int32)
    c = lax.axis_index("c")
    s = lax.axis_index("s")

    pltpu.sync_copy(b1in, bbuf)

    def headm(hh, _hm):
        hd = c * 4 + hh

        def zb(i, _):
            for q in range(4):
                rows[i, pl.ds(q * 16, 16)] = jnp.zeros((16,), jnp.float32)
            return 0
        lax.fori_loop(0, 128, zb, 0)
        for z in range(5):
            pltpu.sync_copy(rows.at[pl.ds(0, 128)],
                            acc.at[pl.ds(s * NB + z * 128, 128)])
        plsc.subcore_barrier()

        def group(g, _):
            row0 = s * RPT + g * 2
            ibuf[pl.ds(0, 16)] = row0 + iota16
            pltpu.async_copy(pk2d.at[ibuf.at[pl.ds(0, 2)]], srci, isem).wait()
            wcp = pltpu.async_copy(w2d.at[hd].at[ibuf.at[pl.ds(0, 2)]],
                                   wbuf, isem)

            def unp(i2, _):
                for j in range(2):
                    v = srci[j, pl.ds(i2 * 16, 16)]
                    dsti[j, pl.ds(i2 * 16, 16)] = v >> 14
                    srci[j, pl.ds(i2 * 16, 16)] = v & 16383
                return 0
            lax.fori_loop(0, 8, unp, 0)

            gcps = [
                pltpu.async_copy(hrows.at[hd].at[srci.at[j]],
                                 rows.at[pl.ds(j * 128, 128)], gsem)
                for j in range(2)
            ]
            wcp.wait()
            for cp in gcps:
                cp.wait()

            for j in range(2):
                def sb(i2, _):
                    wv = wbuf[j, pl.ds(i2 * 16, 16)]
                    for k in range(16):
                        e2 = j * 128 + i2 * 16 + k
                        ws = wv[k]
                        for q in range(4):
                            rows[e2, pl.ds(q * 16, 16)] = (
                                rows[e2, pl.ds(q * 16, 16)] * ws)
                    return 0
                lax.fori_loop(0, 8, sb, 0)

            scps = [
                pltpu.async_copy(rows.at[pl.ds(j * 128, 128)],
                                 acc.at[dsti.at[j]], ssem, add=True)
                for j in range(2)
            ]
            for cp in scps:
                cp.wait()
            return 0
        lax.fori_loop(0, _GROUPS_M, group, 0)
        plsc.subcore_barrier()

        base = hd * 80 + s * 5
        ibuf[pl.ds(0, 16)] = base + iota16
        pltpu.async_copy(den2d.at[ibuf.at[pl.ds(0, 5)]], dbuf, isem).wait()
        bq = [bbuf[hd, pl.ds(q * 16, 16)] for q in range(4)]

        for c2 in range(5):
            pltpu.sync_copy(acc.at[pl.ds(s * NB + c2 * 128, 128)],
                            rows.at[pl.ds(0, 128)])

            def nb_(i8, _):
                dv = dbuf[c2, pl.ds(i8 * 16, 16)]
                for k in range(16):
                    i = i8 * 16 + k
                    dd = dv[k] + 1e-16
                    for q in range(4):
                        v = rows[i, pl.ds(q * 16, 16)] / dd + bq[q]
                        v = jnp.where(v > 0.0, v, jnp.exp(v) - 1.0)
                        rows[i, pl.ds(q * 16, 16)] = v
                return 0
            lax.fori_loop(0, 8, nb_, 0)
            pltpu.sync_copy(rows.at[pl.ds(0, 128)],
                            out1.at[hd, pl.ds(s * NB + c2 * 128, 128)])
        plsc.subcore_barrier()
        return 0
    lax.fori_loop(0, 4, headm, 0)


def _stage_c_body(h1_ref, w2_ref, a2s_ref, a2d_ref, h2ext_ref, a2_ref):
    h1 = jnp.concatenate([h1_ref[k] for k in range(HEADS)], axis=1)
    h2 = jnp.dot(h1, w2_ref[...], preferred_element_type=jnp.float32)
    col = lax.broadcasted_iota(jnp.int32, (NB, 16), 1)
    h2c = h2[:, 0:16]
    h2ext_ref[...] = jnp.where(col < ODIM, h2c,
                               jnp.where(col == ODIM, 1.0, 0.0))
    a2_ref[0, :] = jnp.sum(h2 * a2s_ref[0, :][None, :], axis=1)
    a2_ref[1, :] = jnp.sum(h2 * a2d_ref[0, :][None, :], axis=1)


def _stage_c(out1, W2p, a2sp, a2dp):
    return pl.pallas_call(
        _stage_c_body,
        grid=(NP // NB,),
        in_specs=[
            pl.BlockSpec((HEADS, NB, HID), lambda i: (0, i, 0)),
            pl.BlockSpec((HEADS * HID, 128), lambda i: (0, 0)),
            pl.BlockSpec((1, 128), lambda i: (0, 0)),
            pl.BlockSpec((1, 128), lambda i: (0, 0)),
        ],
        out_specs=[
            pl.BlockSpec((NB, 16), lambda i: (i, 0)),
            pl.BlockSpec((2, NB), lambda i: (0, i)),
        ],
        out_shape=[
            jax.ShapeDtypeStruct((NP, 16), jnp.float32),
            jax.ShapeDtypeStruct((2, NP), jnp.float32),
        ],
    )(out1, W2p, a2sp, a2dp)


_GROUPS2 = EP // 32 // 512


@functools.partial(
    pl.kernel,
    out_type=jax.ShapeDtypeStruct((2, NP, 16), jnp.float32),
    mesh=_MESH,
    scratch_types=[
        pltpu.VMEM((80, 128), jnp.float32),
        pltpu.VMEM((80, 128), jnp.float32),
        pltpu.VMEM((4, 128), jnp.int32),
        pltpu.VMEM((4, 128), jnp.int32),
        pltpu.VMEM((512, 16), jnp.float32),
        pltpu.VMEM((512,), jnp.float32),
        pltpu.VMEM((16,), jnp.int32),
        pltpu.VMEM_SHARED((NP, 16), jnp.float32),
        pltpu.SemaphoreType.DMA,
        pltpu.SemaphoreType.DMA,
        pltpu.SemaphoreType.DMA,
    ],
    compiler_params=_SCPARAMS,
)
def _l2_edge(h2ext, a22d, pk2d, out2p,
             at1, at2, srci, dsti, rows, wbuf, ibuf, acc, gsem, ssem, isem):
    iota16 = jnp.arange(16, dtype=jnp.int32)
    c = lax.axis_index("c")
    s = lax.axis_index("s")
    wid = s * 2 + c

    pltpu.sync_copy(a22d.at[pl.ds(0, 80)], at1)
    pltpu.sync_copy(a22d.at[pl.ds(80, 80)], at2)

    def zb(i, _):
        rows[i, pl.ds(0, 16)] = jnp.zeros((16,), jnp.float32)
        return 0
    lax.fori_loop(0, 128, zb, 0)
    for z in range(5):
        pltpu.sync_copy(rows.at[pl.ds(0, 128)],
                        acc.at[pl.ds(s * NB + z * 128, 128)])
    plsc.subcore_barrier()

    def group(g, _):
        row0 = wid * (_GROUPS2 * 4) + g * 4
        ibuf[pl.ds(0, 16)] = row0 + iota16
        pltpu.async_copy(pk2d.at[ibuf.at[pl.ds(0, 4)]], srci, isem).wait()

        def unp(i2, _):
            for j in range(4):
                v = srci[j, pl.ds(i2 * 16, 16)]
                dsti[j, pl.ds(i2 * 16, 16)] = v >> 14
                srci[j, pl.ds(i2 * 16, 16)] = v & 16383
            return 0
        lax.fori_loop(0, 8, unp, 0)

        gcps = [
            pltpu.async_copy(h2ext.at[srci.at[j]],
                             rows.at[pl.ds(j * 128, 128)], gsem)
            for j in range(4)
        ]
        for j in range(4):
            def wb(i2, _):
                si = srci[j, pl.ds(i2 * 16, 16)]
                di = dsti[j, pl.ds(i2 * 16, 16)]
                e = (plsc.load_gather(at1, [si >> 7, si & 127])
                     + plsc.load_gather(at2, [di >> 7, di & 127]))
                e = jnp.maximum(e, e * 0.2)
                wbuf[pl.ds(j * 128 + i2 * 16, 16)] = jnp.exp(e)
                return 0
            lax.fori_loop(0, 8, wb, 0)
        for cp in gcps:
            cp.wait()

        def sb(i2, _):
            wv = wbuf[pl.ds(i2 * 16, 16)]
            for k in range(16):
                e2 = i2 * 16 + k
                rows[e2, pl.ds(0, 16)] = rows[e2, pl.ds(0, 16)] * wv[k]
            return 0
        lax.fori_loop(0, 32, sb, 0)

        scps = [
            pltpu.async_copy(rows.at[pl.ds(j * 128, 128)],
                             acc.at[dsti.at[j]], ssem, add=True)
            for j in range(4)
        ]
        for cp in scps:
            cp.wait()
        return 0
    lax.fori_loop(0, _GROUPS2, group, 0)
    plsc.subcore_barrier()
    pltpu.sync_copy(acc.at[pl.ds(s * NB, NB)], out2p.at[c, pl.ds(s * NB, NB)])


def _stage_e_body(p_ref, b2_ref, o_ref):
    sm = p_ref[0] + p_ref[1]
    den = sm[:, 3:4] + 1e-16
    lg = sm / den + b2_ref[0, :][None, :]
    col = lax.broadcasted_iota(jnp.int32, (NB, 16), 1)
    valid = col < ODIM
    m = jnp.max(jnp.where(valid, lg, -1e30), axis=1, keepdims=True)
    ex = jnp.where(valid, jnp.exp(lg - m), 0.0)
    lse = jnp.log(jnp.sum(ex, axis=1, keepdims=True))
    o_ref[...] = lg - m - lse


def _stage_e(out2p, b2p):
    return pl.pallas_call(
        _stage_e_body,
        grid=(NP // NB,),
        in_specs=[
            pl.BlockSpec((2, NB, 16), lambda i: (0, i, 0)),
            pl.BlockSpec((1, 16), lambda i: (0, 0)),
        ],
        out_specs=pl.BlockSpec((NB, 16), lambda i: (i, 0)),
        out_shape=jax.ShapeDtypeStruct((NP, 16), jnp.float32),
    )(out2p, b2p)


def kernel(x, edge_index, W1, a_src1, a_dst1, b1, W2, a_src2, a_dst2, b2):
    ei = edge_index.astype(jnp.int32)
    loop = jnp.arange(N, dtype=jnp.int32)
    padv = jnp.full((EP - E - N,), NP - 1, jnp.int32)
    src = jnp.concatenate([ei[0], loop, padv])
    dst = jnp.concatenate([ei[1], loop, padv])
    pk2d = (src | (dst << 14)).reshape(EROWS, 128)

    xp = jnp.pad(x, ((0, NP - N), (0, 0)))
    W2p = jnp.pad(W2, ((0, 0), (0, 128 - ODIM)))
    a2sp = jnp.pad(a_src2[0], ((0, 0), (0, 128 - ODIM)))
    a2dp = jnp.pad(a_dst2[0], ((0, 0), (0, 128 - ODIM)))
    b2p = jnp.pad(b2, (0, 16 - ODIM)).reshape(1, 16)
    b1in = jnp.pad(b1.reshape(HEADS, HID), ((0, 0), (0, 128 - HID)))

    hrows, aso, ado = _stage_a(xp, W1, a_src1, a_dst1)
    aso2 = aso.reshape(HEADS * 80, 128)
    ado2 = ado.reshape(HEADS * 80, 128)
    w2d, den = _w_kernel(aso2, ado2, pk2d)
    den2d = den.reshape(HEADS * 80, 128)
    out1 = _m_kernel(hrows, pk2d, w2d, den2d, b1in)
    h2ext, a2 = _stage_c(out1, W2p, a2sp, a2dp)
    a22d = a2.reshape(160, 128)
    out2p = _l2_edge(h2ext, a22d, pk2d)
    res = _stage_e(out2p, b2p)
    return res[:N, :ODIM]


# speedup vs baseline: 19.5992x; 1.5977x over previous
import functools

import jax
import jax.numpy as jnp
from jax import lax
from jax.experimental import pallas as pl
from jax.experimental.pallas import tpu as pltpu
from jax.experimental.pallas import tpu_sc as plsc

N = 10000
NP = 10240
E = 320000
EP = 344064
EROWS = EP // 128
RPT = EROWS // 16
HEADS = 8
HID = 64
ODIM = 3
NB = 640

_SCPARAMS = pltpu.CompilerParams(needs_layout_passes=False,
                                 use_tc_tiling_on_sc=False)
_MESH = plsc.VectorSubcoreMesh(core_axis_name="c", subcore_axis_name="s")


def _stage_a_body(x_ref, w_ref, asr_ref, adr_ref, hh_ref, aso_ref, ado_ref):
    h = jnp.dot(x_ref[...], w_ref[...], preferred_element_type=jnp.float32)
    for hd in range(HEADS):
        hh = h[:, hd * HID:(hd + 1) * HID]
        hh_ref[hd] = hh
        aso_ref[hd, :] = jnp.sum(hh * asr_ref[0, hd, :][None, :], axis=1)
        ado_ref[hd, :] = jnp.sum(hh * adr_ref[0, hd, :][None, :], axis=1)


def _stage_a(xp, W1, a_src1, a_dst1):
    return pl.pallas_call(
        _stage_a_body,
        grid=(NP // NB,),
        in_specs=[
            pl.BlockSpec((NB, 128), lambda i: (i, 0)),
            pl.BlockSpec((128, HEADS * HID), lambda i: (0, 0)),
            pl.BlockSpec((1, HEADS, HID), lambda i: (0, 0, 0)),
            pl.BlockSpec((1, HEADS, HID), lambda i: (0, 0, 0)),
        ],
        out_specs=[
            pl.BlockSpec((HEADS, NB, HID), lambda i: (0, i, 0)),
            pl.BlockSpec((HEADS, NB), lambda i: (0, i)),
            pl.BlockSpec((HEADS, NB), lambda i: (0, i)),
        ],
        out_shape=[
            jax.ShapeDtypeStruct((HEADS, NP, HID), jnp.float32),
            jax.ShapeDtypeStruct((HEADS, NP), jnp.float32),
            jax.ShapeDtypeStruct((HEADS, NP), jnp.float32),
        ],
    )(xp, W1, a_src1, a_dst1)


_GROUPS_W = RPT // 8


@functools.partial(
    pl.kernel,
    out_type=[
        jax.ShapeDtypeStruct((HEADS, EROWS, 128), jnp.float32),
        jax.ShapeDtypeStruct((HEADS, NP), jnp.float32),
    ],
    mesh=_MESH,
    scratch_types=[
        pltpu.VMEM((80, 128), jnp.float32),
        pltpu.VMEM((80, 128), jnp.float32),
        pltpu.VMEM((8, 128), jnp.int32),
        pltpu.VMEM((8, 128), jnp.int32),
        pltpu.VMEM((8, 128), jnp.float32),
        pltpu.VMEM((1024, 16), jnp.float32),
        pltpu.VMEM((640, 16), jnp.float32),
        pltpu.VMEM((640,), jnp.float32),
        pltpu.VMEM((16,), jnp.int32),
        pltpu.VMEM((80,), jnp.int32),
        pltpu.VMEM_SHARED((NP, 16), jnp.float32),
        pltpu.SemaphoreType.DMA,
        pltpu.SemaphoreType.DMA,
    ],
    compiler_params=_SCPARAMS,
)
def _w_kernel(aso2, ado2, pk2d, wout, den,
              at1, at2, srci, dsti, wrow, rden, dbuf, den1, ibuf, ib80,
              accd, isem, ssem):
    iota16 = jnp.arange(16, dtype=jnp.int32)
    c = lax.axis_index("c")
    s = lax.axis_index("s")

    def zb(i, _):
        rden[i, pl.ds(0, 16)] = jnp.zeros((16,), jnp.float32)
        return 0
    lax.fori_loop(0, 128, zb, 0)
    for z in range(5):
        pltpu.sync_copy(rden.at[pl.ds(0, 128)],
                        accd.at[pl.ds(s * NB + z * 128, 128)])
    plsc.subcore_barrier()

    def headw(hh, _hw):
        hd = c * 4 + hh
        base80 = hd * 80
        for z in range(5):
            ib80[pl.ds(z * 16, 16)] = base80 + z * 16 + iota16
        pltpu.async_copy(aso2.at[ib80], at1, isem).wait()
        pltpu.async_copy(ado2.at[ib80], at2, isem).wait()

        def group(g, _):
            row0 = s * RPT + g * 8
            ibuf[pl.ds(0, 16)] = row0 + iota16
            pltpu.async_copy(pk2d.at[ibuf.at[pl.ds(0, 8)]], srci, isem).wait()

            def unp(i2, _):
                for j in range(8):
                    v = srci[j, pl.ds(i2 * 16, 16)]
                    dsti[j, pl.ds(i2 * 16, 16)] = v >> 14
                    srci[j, pl.ds(i2 * 16, 16)] = v & 16383
                return 0
            lax.fori_loop(0, 8, unp, 0)

            for j in range(8):
                def wb(i2, _):
                    si = srci[j, pl.ds(i2 * 16, 16)]
                    di = dsti[j, pl.ds(i2 * 16, 16)]
                    a = plsc.load_gather(at1, [si >> 7, si & 127])
                    b = plsc.load_gather(at2, [di >> 7, di & 127])
                    e = a + b
                    e = jnp.maximum(e, e * 0.2)
                    wv = jnp.exp(e)
                    wrow[j, pl.ds(i2 * 16, 16)] = wv

                    for k in range(16):
                        e2 = j * 128 + i2 * 16 + k
                        rden[e2, pl.ds(0, 16)] = jnp.where(
                            iota16 == hh, wv[k], 0.0)
                    return 0
                lax.fori_loop(0, 8, wb, 0)

            pltpu.sync_copy(wrow, wout.at[hd, pl.ds(row0, 8)])
            scps = [
                pltpu.async_copy(rden.at[pl.ds(j * 128, 128)],
                                 accd.at[dsti.at[j]], ssem, add=True)
                for j in range(8)
            ]
            for cp in scps:
                cp.wait()
            return 0
        lax.fori_loop(0, _GROUPS_W, group, 0)
        return 0
    lax.fori_loop(0, 4, headw, 0)

    plsc.subcore_barrier()
    pltpu.sync_copy(accd.at[pl.ds(s * NB, NB)], dbuf)
    def headx(hh, _hx):
        hd = c * 4 + hh
        hv = jnp.zeros((16,), jnp.int32) + hh

        def dx(i, _):
            iv = i * 16 + iota16
            den1[pl.ds(i * 16, 16)] = plsc.load_gather(dbuf, [iv, hv])
            return 0
        lax.fori_loop(0, 40, dx, 0)
        pltpu.sync_copy(den1, den.at[hd, pl.ds(s * NB, NB)])
        return 0
    lax.fori_loop(0, 4, headx, 0)


_GROUPS_M = RPT // 4


@functools.partial(
    pl.kernel,
    out_type=jax.ShapeDtypeStruct((HEADS, NP, HID), jnp.float32),
    mesh=_MESH,
    scratch_types=[
        pltpu.VMEM((512, HID), jnp.float32),
        pltpu.VMEM((2, 4, 128), jnp.int32),
        pltpu.VMEM((2, 4, 128), jnp.int32),
        pltpu.VMEM((2, 4, 128), jnp.float32),
        pltpu.VMEM((5, 128), jnp.float32),
        pltpu.VMEM((8, 128), jnp.float32),
        pltpu.VMEM((2, 16), jnp.int32),
        pltpu.VMEM_SHARED((NP, HID), jnp.float32),
        pltpu.SemaphoreType.DMA,
        pltpu.SemaphoreType.DMA,
        pltpu.SemaphoreType.DMA,
    ],
    compiler_params=_SCPARAMS,
)
def _m_kernel(hrows, pk2d, w2d, den2d, b1in, out1,
              rows, srci, dsti, wbuf, dbuf, bbuf, ibuf,
              acc, gsem, ssem, isem):
    iota16 = jnp.arange(16, dtype=jnp.int32)
    c = lax.axis_index("c")
    s = lax.axis_index("s")

    pltpu.sync_copy(b1in, bbuf)

    def headm(hh, _hm):
        hd = c * 4 + hh

        def zb(i, _):
            for q in range(4):
                rows[i, pl.ds(q * 16, 16)] = jnp.zeros((16,), jnp.float32)
            return 0
        lax.fori_loop(0, 128, zb, 0)
        for z in range(5):
            pltpu.sync_copy(rows.at[pl.ds(0, 128)],
                            acc.at[pl.ds(s * NB + z * 128, 128)])
        plsc.subcore_barrier()

        def fetch(gnum, b):
            base = jnp.where(gnum < _GROUPS_M, s * RPT + gnum * 4, 0)
            ibuf[b, pl.ds(0, 16)] = base + iota16
            pltpu.async_copy(pk2d.at[ibuf.at[b, pl.ds(0, 4)]],
                             srci.at[b], isem)
            pltpu.async_copy(w2d.at[hd].at[ibuf.at[b, pl.ds(0, 4)]],
                             wbuf.at[b], isem)

        fetch(jnp.int32(0), 0)

        def pair(g, _):
            for b in range(2):
                gnum = g * 2 + b
                pltpu.make_async_copy(pk2d.at[ibuf.at[b, pl.ds(0, 4)]],
                                      srci.at[b], isem).wait()
                pltpu.make_async_copy(w2d.at[hd].at[ibuf.at[b, pl.ds(0, 4)]],
                                      wbuf.at[b], isem).wait()

                def unp(i2, _):
                    for j in range(4):
                        v = srci[b, j, pl.ds(i2 * 16, 16)]
                        dsti[b, j, pl.ds(i2 * 16, 16)] = v >> 14
                        srci[b, j, pl.ds(i2 * 16, 16)] = v & 16383
                    return 0
                lax.fori_loop(0, 8, unp, 0)

                def drain(_n=None):
                    for j in range(4):
                        pltpu.make_async_copy(
                            rows.at[pl.ds(j * 128, 128)],
                            acc.at[dsti.at[b, j]], ssem).wait()
                if b == 0:
                    @pl.when(g > 0)
                    def _():
                        drain()
                else:
                    drain()

                gcps = [
                    pltpu.async_copy(hrows.at[hd].at[srci.at[b, j]],
                                     rows.at[pl.ds(j * 128, 128)], gsem)
                    for j in range(4)
                ]
                fetch(gnum + 1, 1 - b)

                for j in range(4):
                    gcps[j].wait()

                    def sb(i2, _):
                        wv = wbuf[b, j, pl.ds(i2 * 16, 16)]
                        for k in range(16):
                            e2 = j * 128 + i2 * 16 + k
                            ws = wv[k]
                            for q in range(4):
                                rows[e2, pl.ds(q * 16, 16)] = (
                                    rows[e2, pl.ds(q * 16, 16)] * ws)
                        return 0
                    lax.fori_loop(0, 8, sb, 0)
                    pltpu.async_copy(rows.at[pl.ds(j * 128, 128)],
                                     acc.at[dsti.at[b, j]], ssem, add=True)
            return 0
        lax.fori_loop(0, _GROUPS_M // 2, pair, 0)

        for j in range(4):
            pltpu.make_async_copy(rows.at[pl.ds(j * 128, 128)],
                                  acc.at[dsti.at[1, j]], ssem).wait()
        pltpu.make_async_copy(pk2d.at[ibuf.at[0, pl.ds(0, 4)]],
                              srci.at[0], isem).wait()
        pltpu.make_async_copy(w2d.at[hd].at[ibuf.at[0, pl.ds(0, 4)]],
                              wbuf.at[0], isem).wait()
        plsc.subcore_barrier()

        base = hd * 80 + s * 5
        ibuf[0, pl.ds(0, 16)] = base + iota16
        pltpu.async_copy(den2d.at[ibuf.at[0, pl.ds(0, 5)]], dbuf, isem).wait()
        bq = [bbuf[hd, pl.ds(q * 16, 16)] for q in range(4)]

        for c2 in range(5):
            pltpu.sync_copy(acc.at[pl.ds(s * NB + c2 * 128, 128)],
                            rows.at[pl.ds(0, 128)])

            def nb_(i8, _):
                dv = dbuf[c2, pl.ds(i8 * 16, 16)]
                for k in range(16):
                    i = i8 * 16 + k
                    dd = dv[k] + 1e-16
                    for q in range(4):
                        v = rows[i, pl.ds(q * 16, 16)] / dd + bq[q]
                        v = jnp.where(v > 0.0, v, jnp.exp(v) - 1.0)
                        rows[i, pl.ds(q * 16, 16)] = v
                return 0
            lax.fori_loop(0, 8, nb_, 0)
            pltpu.sync_copy(rows.at[pl.ds(0, 128)],
                            out1.at[hd, pl.ds(s * NB + c2 * 128, 128)])
        plsc.subcore_barrier()
        return 0
    lax.fori_loop(0, 4, headm, 0)


def _stage_c_body(h1_ref, w2_ref, a2s_ref, a2d_ref, h2ext_ref, a2_ref):
    h1 = jnp.concatenate([h1_ref[k] for k in range(HEADS)], axis=1)
    h2 = jnp.dot(h1, w2_ref[...], preferred_element_type=jnp.float32)
    col = lax.broadcasted_iota(jnp.int32, (NB, 16), 1)
    h2c = h2[:, 0:16]
    h2ext_ref[...] = jnp.where(col < ODIM, h2c,
                               jnp.where(col == ODIM, 1.0, 0.0))
    a2_ref[0, :] = jnp.sum(h2 * a2s_ref[0, :][None, :], axis=1)
    a2_ref[1, :] = jnp.sum(h2 * a2d_ref[0, :][None, :], axis=1)


def _stage_c(out1, W2p, a2sp, a2dp):
    return pl.pallas_call(
        _stage_c_body,
        grid=(NP // NB,),
        in_specs=[
            pl.BlockSpec((HEADS, NB, HID), lambda i: (0, i, 0)),
            pl.BlockSpec((HEADS * HID, 128), lambda i: (0, 0)),
            pl.BlockSpec((1, 128), lambda i: (0, 0)),
            pl.BlockSpec((1, 128), lambda i: (0, 0)),
        ],
        out_specs=[
            pl.BlockSpec((NB, 16), lambda i: (i, 0)),
            pl.BlockSpec((2, NB), lambda i: (0, i)),
        ],
        out_shape=[
            jax.ShapeDtypeStruct((NP, 16), jnp.float32),
            jax.ShapeDtypeStruct((2, NP), jnp.float32),
        ],
    )(out1, W2p, a2sp, a2dp)


_GROUPS2 = EP // 32 // 512


@functools.partial(
    pl.kernel,
    out_type=jax.ShapeDtypeStruct((2, NP, 16), jnp.float32),
    mesh=_MESH,
    scratch_types=[
        pltpu.VMEM((80, 128), jnp.float32),
        pltpu.VMEM((80, 128), jnp.float32),
        pltpu.VMEM((4, 128), jnp.int32),
        pltpu.VMEM((4, 128), jnp.int32),
        pltpu.VMEM((512, 16), jnp.float32),
        pltpu.VMEM((512,), jnp.float32),
        pltpu.VMEM((16,), jnp.int32),
        pltpu.VMEM_SHARED((NP, 16), jnp.float32),
        pltpu.SemaphoreType.DMA,
        pltpu.SemaphoreType.DMA,
        pltpu.SemaphoreType.DMA,
    ],
    compiler_params=_SCPARAMS,
)
def _l2_edge(h2ext, a22d, pk2d, out2p,
             at1, at2, srci, dsti, rows, wbuf, ibuf, acc, gsem, ssem, isem):
    iota16 = jnp.arange(16, dtype=jnp.int32)
    c = lax.axis_index("c")
    s = lax.axis_index("s")
    wid = s * 2 + c

    pltpu.sync_copy(a22d.at[pl.ds(0, 80)], at1)
    pltpu.sync_copy(a22d.at[pl.ds(80, 80)], at2)

    def zb(i, _):
        rows[i, pl.ds(0, 16)] = jnp.zeros((16,), jnp.float32)
        return 0
    lax.fori_loop(0, 128, zb, 0)
    for z in range(5):
        pltpu.sync_copy(rows.at[pl.ds(0, 128)],
                        acc.at[pl.ds(s * NB + z * 128, 128)])
    plsc.subcore_barrier()

    def group(g, _):
        row0 = wid * (_GROUPS2 * 4) + g * 4
        ibuf[pl.ds(0, 16)] = row0 + iota16
        pltpu.async_copy(pk2d.at[ibuf.at[pl.ds(0, 4)]], srci, isem).wait()

        def unp(i2, _):
            for j in range(4):
                v = srci[j, pl.ds(i2 * 16, 16)]
                dsti[j, pl.ds(i2 * 16, 16)] = v >> 14
                srci[j, pl.ds(i2 * 16, 16)] = v & 16383
            return 0
        lax.fori_loop(0, 8, unp, 0)

        gcps = [
            pltpu.async_copy(h2ext.at[srci.at[j]],
                             rows.at[pl.ds(j * 128, 128)], gsem)
            for j in range(4)
        ]
        for j in range(4):
            def wb(i2, _):
                si = srci[j, pl.ds(i2 * 16, 16)]
                di = dsti[j, pl.ds(i2 * 16, 16)]
                e = (plsc.load_gather(at1, [si >> 7, si & 127])
                     + plsc.load_gather(at2, [di >> 7, di & 127]))
                e = jnp.maximum(e, e * 0.2)
                wbuf[pl.ds(j * 128 + i2 * 16, 16)] = jnp.exp(e)
                return 0
            lax.fori_loop(0, 8, wb, 0)
        for cp in gcps:
            cp.wait()

        def sb(i2, _):
            wv = wbuf[pl.ds(i2 * 16, 16)]
            for k in range(16):
                e2 = i2 * 16 + k
                rows[e2, pl.ds(0, 16)] = rows[e2, pl.ds(0, 16)] * wv[k]
            return 0
        lax.fori_loop(0, 32, sb, 0)

        scps = [
            pltpu.async_copy(rows.at[pl.ds(j * 128, 128)],
                             acc.at[dsti.at[j]], ssem, add=True)
            for j in range(4)
        ]
        for cp in scps:
            cp.wait()
        return 0
    lax.fori_loop(0, _GROUPS2, group, 0)
    plsc.subcore_barrier()
    pltpu.sync_copy(acc.at[pl.ds(s * NB, NB)], out2p.at[c, pl.ds(s * NB, NB)])


def _stage_e_body(p_ref, b2_ref, o_ref):
    sm = p_ref[0] + p_ref[1]
    den = sm[:, 3:4] + 1e-16
    lg = sm / den + b2_ref[0, :][None, :]
    col = lax.broadcasted_iota(jnp.int32, (NB, 16), 1)
    valid = col < ODIM
    m = jnp.max(jnp.where(valid, lg, -1e30), axis=1, keepdims=True)
    ex = jnp.where(valid, jnp.exp(lg - m), 0.0)
    lse = jnp.log(jnp.sum(ex, axis=1, keepdims=True))
    o_ref[...] = lg - m - lse


def _stage_e(out2p, b2p):
    return pl.pallas_call(
        _stage_e_body,
        grid=(NP // NB,),
        in_specs=[
            pl.BlockSpec((2, NB, 16), lambda i: (0, i, 0)),
            pl.BlockSpec((1, 16), lambda i: (0, 0)),
        ],
        out_specs=pl.BlockSpec((NB, 16), lambda i: (i, 0)),
        out_shape=jax.ShapeDtypeStruct((NP, 16), jnp.float32),
    )(out2p, b2p)


def kernel(x, edge_index, W1, a_src1, a_dst1, b1, W2, a_src2, a_dst2, b2):
    ei = edge_index.astype(jnp.int32)
    loop = jnp.arange(N, dtype=jnp.int32)
    padv = jnp.full((EP - E - N,), NP - 1, jnp.int32)
    src = jnp.concatenate([ei[0], loop, padv])
    dst = jnp.concatenate([ei[1], loop, padv])
    pk2d = (src | (dst << 14)).reshape(EROWS, 128)

    xp = jnp.pad(x, ((0, NP - N), (0, 0)))
    W2p = jnp.pad(W2, ((0, 0), (0, 128 - ODIM)))
    a2sp = jnp.pad(a_src2[0], ((0, 0), (0, 128 - ODIM)))
    a2dp = jnp.pad(a_dst2[0], ((0, 0), (0, 128 - ODIM)))
    b2p = jnp.pad(b2, (0, 16 - ODIM)).reshape(1, 16)
    b1in = jnp.pad(b1.reshape(HEADS, HID), ((0, 0), (0, 128 - HID)))

    hrows, aso, ado = _stage_a(xp, W1, a_src1, a_dst1)
    aso2 = aso.reshape(HEADS * 80, 128)
    ado2 = ado.reshape(HEADS * 80, 128)
    w2d, den = _w_kernel(aso2, ado2, pk2d)
    den2d = den.reshape(HEADS * 80, 128)
    out1 = _m_kernel(hrows, pk2d, w2d, den2d, b1in)
    h2ext, a2 = _stage_c(out1, W2p, a2sp, a2dp)
    a22d = a2.reshape(160, 128)
    out2p = _l2_edge(h2ext, a22d, pk2d)
    res = _stage_e(out2p, b2p)
    return res[:N, :ODIM]


# speedup vs baseline: 22.4126x; 1.1435x over previous
import functools

import jax
import jax.numpy as jnp
from jax import lax
from jax.experimental import pallas as pl
from jax.experimental.pallas import tpu as pltpu
from jax.experimental.pallas import tpu_sc as plsc

N = 10000
NP = 10240
E = 320000
EP = 344064
EROWS = EP // 128
RPT = EROWS // 16
HEADS = 8
HID = 64
ODIM = 3
NB = 640

_SCPARAMS = pltpu.CompilerParams(needs_layout_passes=False,
                                 use_tc_tiling_on_sc=False)
_MESH = plsc.VectorSubcoreMesh(core_axis_name="c", subcore_axis_name="s")


def _stage_a_body(x_ref, w_ref, asr_ref, adr_ref, hh_ref, hb_ref,
                  aso_ref, ado_ref):
    h = jnp.dot(x_ref[...], w_ref[...], preferred_element_type=jnp.float32)
    for hd in range(HEADS):
        hh = h[:, hd * HID:(hd + 1) * HID]
        hh_ref[hd] = hh
        hb_ref[hd] = hh.astype(jnp.bfloat16)
        aso_ref[hd, :] = jnp.sum(hh * asr_ref[0, hd, :][None, :], axis=1)
        ado_ref[hd, :] = jnp.sum(hh * adr_ref[0, hd, :][None, :], axis=1)


def _stage_a(xp, W1, a_src1, a_dst1):
    return pl.pallas_call(
        _stage_a_body,
        grid=(NP // NB,),
        in_specs=[
            pl.BlockSpec((NB, 128), lambda i: (i, 0)),
            pl.BlockSpec((128, HEADS * HID), lambda i: (0, 0)),
            pl.BlockSpec((1, HEADS, HID), lambda i: (0, 0, 0)),
            pl.BlockSpec((1, HEADS, HID), lambda i: (0, 0, 0)),
        ],
        out_specs=[
            pl.BlockSpec((HEADS, NB, HID), lambda i: (0, i, 0)),
            pl.BlockSpec((HEADS, NB, HID), lambda i: (0, i, 0)),
            pl.BlockSpec((HEADS, NB), lambda i: (0, i)),
            pl.BlockSpec((HEADS, NB), lambda i: (0, i)),
        ],
        out_shape=[
            jax.ShapeDtypeStruct((HEADS, NP, HID), jnp.float32),
            jax.ShapeDtypeStruct((HEADS, NP, HID), jnp.bfloat16),
            jax.ShapeDtypeStruct((HEADS, NP), jnp.float32),
            jax.ShapeDtypeStruct((HEADS, NP), jnp.float32),
        ],
    )(xp, W1, a_src1, a_dst1)


_GROUPS_W = RPT // 8


@functools.partial(
    pl.kernel,
    out_type=[
        jax.ShapeDtypeStruct((HEADS, EROWS, 128), jnp.float32),
        jax.ShapeDtypeStruct((HEADS, NP), jnp.float32),
    ],
    mesh=_MESH,
    scratch_types=[
        pltpu.VMEM((80, 128), jnp.float32),
        pltpu.VMEM((80, 128), jnp.float32),
        pltpu.VMEM((8, 128), jnp.int32),
        pltpu.VMEM((8, 128), jnp.int32),
        pltpu.VMEM((8, 128), jnp.float32),
        pltpu.VMEM((1024, 16), jnp.float32),
        pltpu.VMEM((640, 16), jnp.float32),
        pltpu.VMEM((640,), jnp.float32),
        pltpu.VMEM((16,), jnp.int32),
        pltpu.VMEM((80,), jnp.int32),
        pltpu.VMEM_SHARED((NP, 16), jnp.float32),
        pltpu.SemaphoreType.DMA,
        pltpu.SemaphoreType.DMA,
    ],
    compiler_params=_SCPARAMS,
)
def _w_kernel(aso2, ado2, pk2d, wout, den,
              at1, at2, srci, dsti, wrow, rden, dbuf, den1, ibuf, ib80,
              accd, isem, ssem):
    iota16 = jnp.arange(16, dtype=jnp.int32)
    c = lax.axis_index("c")
    s = lax.axis_index("s")

    def zb(i, _):
        rden[i, pl.ds(0, 16)] = jnp.zeros((16,), jnp.float32)
        return 0
    lax.fori_loop(0, 128, zb, 0)
    for z in range(5):
        pltpu.sync_copy(rden.at[pl.ds(0, 128)],
                        accd.at[pl.ds(s * NB + z * 128, 128)])
    plsc.subcore_barrier()

    def headw(hh, _hw):
        hd = c * 4 + hh
        base80 = hd * 80
        for z in range(5):
            ib80[pl.ds(z * 16, 16)] = base80 + z * 16 + iota16
        pltpu.async_copy(aso2.at[ib80], at1, isem).wait()
        pltpu.async_copy(ado2.at[ib80], at2, isem).wait()

        def group(g, _):
            row0 = s * RPT + g * 8
            ibuf[pl.ds(0, 16)] = row0 + iota16
            pltpu.async_copy(pk2d.at[ibuf.at[pl.ds(0, 8)]], srci, isem).wait()

            def unp(i2, _):
                for j in range(8):
                    v = srci[j, pl.ds(i2 * 16, 16)]
                    dsti[j, pl.ds(i2 * 16, 16)] = v >> 14
                    srci[j, pl.ds(i2 * 16, 16)] = v & 16383
                return 0
            lax.fori_loop(0, 8, unp, 0)

            for j in range(8):
                def wb(i2, _):
                    si = srci[j, pl.ds(i2 * 16, 16)]
                    di = dsti[j, pl.ds(i2 * 16, 16)]
                    a = plsc.load_gather(at1, [si >> 7, si & 127])
                    b = plsc.load_gather(at2, [di >> 7, di & 127])
                    e = a + b
                    e = jnp.maximum(e, e * 0.2)
                    wv = jnp.exp(e)
                    wrow[j, pl.ds(i2 * 16, 16)] = wv

                    for k in range(16):
                        e2 = j * 128 + i2 * 16 + k
                        rden[e2, pl.ds(0, 16)] = jnp.where(
                            iota16 == hh, wv[k], 0.0)
                    return 0
                lax.fori_loop(0, 8, wb, 0)

            pltpu.sync_copy(wrow, wout.at[hd, pl.ds(row0, 8)])
            scps = [
                pltpu.async_copy(rden.at[pl.ds(j * 128, 128)],
                                 accd.at[dsti.at[j]], ssem, add=True)
                for j in range(8)
            ]
            for cp in scps:
                cp.wait()
            return 0
        lax.fori_loop(0, _GROUPS_W, group, 0)
        return 0
    lax.fori_loop(0, 4, headw, 0)

    plsc.subcore_barrier()
    pltpu.sync_copy(accd.at[pl.ds(s * NB, NB)], dbuf)
    def headx(hh, _hx):
        hd = c * 4 + hh
        hv = jnp.zeros((16,), jnp.int32) + hh

        def dx(i, _):
            iv = i * 16 + iota16
            den1[pl.ds(i * 16, 16)] = plsc.load_gather(dbuf, [iv, hv])
            return 0
        lax.fori_loop(0, 40, dx, 0)
        pltpu.sync_copy(den1, den.at[hd, pl.ds(s * NB, NB)])
        return 0
    lax.fori_loop(0, 4, headx, 0)


_GROUPS_M = RPT // 2


@functools.partial(
    pl.kernel,
    out_type=jax.ShapeDtypeStruct((HEADS, NP, HID), jnp.float32),
    mesh=_MESH,
    scratch_types=[
        pltpu.VMEM((256, HID), jnp.float32),
        pltpu.VMEM((256, 32), jnp.int32),
        pltpu.VMEM((2, 2, 128), jnp.int32),
        pltpu.VMEM((2, 2, 128), jnp.int32),
        pltpu.VMEM((2, 2, 128), jnp.float32),
        pltpu.VMEM((5, 128), jnp.float32),
        pltpu.VMEM((8, 128), jnp.float32),
        pltpu.VMEM((2, 16), jnp.int32),
        pltpu.VMEM_SHARED((NP, HID), jnp.float32),
        pltpu.SemaphoreType.DMA,
        pltpu.SemaphoreType.DMA,
        pltpu.SemaphoreType.DMA,
    ],
    compiler_params=_SCPARAMS,
)
def _m_kernel(hbi, pk2d, w2d, den2d, b1in, out1,
              rows, rbi, srci, dsti, wbuf, dbuf, bbuf, ibuf,
              acc, gsem, ssem, isem):
    iota16 = jnp.arange(16, dtype=jnp.int32)
    c = lax.axis_index("c")
    s = lax.axis_index("s")

    pltpu.sync_copy(b1in, bbuf)

    def headm(hh, _hm):
        hd = c * 4 + hh

        def zb(i, _):
            for q in range(4):
                rows[i, pl.ds(q * 16, 16)] = jnp.zeros((16,), jnp.float32)
            return 0
        lax.fori_loop(0, 128, zb, 0)
        for z in range(5):
            pltpu.sync_copy(rows.at[pl.ds(0, 128)],
                            acc.at[pl.ds(s * NB + z * 128, 128)])
        plsc.subcore_barrier()

        def fetch(gnum, b):
            base = jnp.where(gnum < _GROUPS_M, s * RPT + gnum * 2, 0)
            ibuf[b, pl.ds(0, 16)] = base + iota16
            pltpu.async_copy(pk2d.at[ibuf.at[b, pl.ds(0, 2)]],
                             srci.at[b], isem)
            pltpu.async_copy(w2d.at[hd].at[ibuf.at[b, pl.ds(0, 2)]],
                             wbuf.at[b], isem)

        fetch(jnp.int32(0), 0)

        def pair(g, _):
            for b in range(2):
                gnum = g * 2 + b
                pltpu.make_async_copy(pk2d.at[ibuf.at[b, pl.ds(0, 2)]],
                                      srci.at[b], isem).wait()
                pltpu.make_async_copy(w2d.at[hd].at[ibuf.at[b, pl.ds(0, 2)]],
                                      wbuf.at[b], isem).wait()

                def unp(i2, _):
                    for j in range(2):
                        v = srci[b, j, pl.ds(i2 * 16, 16)]
                        dsti[b, j, pl.ds(i2 * 16, 16)] = v >> 14
                        srci[b, j, pl.ds(i2 * 16, 16)] = v & 16383
                    return 0
                lax.fori_loop(0, 8, unp, 0)

                gcps = [
                    pltpu.async_copy(hbi.at[hd].at[srci.at[b, j]],
                                     rbi.at[pl.ds(j * 128, 128)], gsem)
                    for j in range(2)
                ]
                fetch(gnum + 1, 1 - b)

                for j in range(2):
                    gcps[j].wait()

                    def drain(jj=j):
                        pltpu.make_async_copy(
                            rows.at[pl.ds(jj * 128, 128)],
                            acc.at[dsti.at[b, jj]], ssem).wait()
                    if b == 0:
                        @pl.when(g > 0)
                        def _():
                            drain()
                    else:
                        drain()

                    def sb(i2, _):
                        wv = wbuf[b, j, pl.ds(i2 * 16, 16)]
                        for k in range(16):
                            e2 = j * 128 + i2 * 16 + k
                            ws = wv[k]
                            for qq in range(2):
                                wd = rbi[e2, pl.ds(qq * 16, 16)]
                                ev = plsc.bitcast(wd << 16, jnp.float32)
                                od = plsc.bitcast(
                                    wd & jnp.int32(-65536), jnp.float32)
                                rows[e2, pl.ds(qq * 32, 16)] = ev * ws
                                rows[e2, pl.ds(qq * 32 + 16, 16)] = od * ws
                        return 0
                    lax.fori_loop(0, 8, sb, 0)
                    pltpu.async_copy(rows.at[pl.ds(j * 128, 128)],
                                     acc.at[dsti.at[b, j]], ssem, add=True)
            return 0
        lax.fori_loop(0, _GROUPS_M // 2, pair, 0)

        for j in range(2):
            pltpu.make_async_copy(rows.at[pl.ds(j * 128, 128)],
                                  acc.at[dsti.at[1, j]], ssem).wait()
        pltpu.make_async_copy(pk2d.at[ibuf.at[0, pl.ds(0, 2)]],
                              srci.at[0], isem).wait()
        pltpu.make_async_copy(w2d.at[hd].at[ibuf.at[0, pl.ds(0, 2)]],
                              wbuf.at[0], isem).wait()
        plsc.subcore_barrier()

        base = hd * 80 + s * 5
        ibuf[0, pl.ds(0, 16)] = base + iota16
        pltpu.async_copy(den2d.at[ibuf.at[0, pl.ds(0, 5)]], dbuf, isem).wait()
        bq = [bbuf[hd, pl.ds(q * 16, 16)] for q in range(4)]

        for c2 in range(5):
            pltpu.sync_copy(acc.at[pl.ds(s * NB + c2 * 128, 128)],
                            rows.at[pl.ds(0, 128)])

            def nb_(i8, _):
                dv = dbuf[c2, pl.ds(i8 * 16, 16)]
                for k in range(16):
                    i = i8 * 16 + k
                    dd = dv[k] + 1e-16
                    for q in range(4):
                        v = rows[i, pl.ds(q * 16, 16)] / dd + bq[q]
                        v = jnp.where(v > 0.0, v, jnp.exp(v) - 1.0)
                        rows[i, pl.ds(q * 16, 16)] = v
                return 0
            lax.fori_loop(0, 8, nb_, 0)
            pltpu.sync_copy(rows.at[pl.ds(0, 128)],
                            out1.at[hd, pl.ds(s * NB + c2 * 128, 128)])
        plsc.subcore_barrier()
        return 0
    lax.fori_loop(0, 4, headm, 0)


def _stage_c_body(h1_ref, w2_ref, a2s_ref, a2d_ref, h2ext_ref, a2_ref):
    h1 = jnp.concatenate([h1_ref[k] for k in range(HEADS)], axis=1)
    h2 = jnp.dot(h1, w2_ref[...], preferred_element_type=jnp.float32)
    col = lax.broadcasted_iota(jnp.int32, (NB, 16), 1)
    h2c = h2[:, 0:16]
    h2ext_ref[...] = jnp.where(col < ODIM, h2c,
                               jnp.where(col == ODIM, 1.0, 0.0))
    a2_ref[0, :] = jnp.sum(h2 * a2s_ref[0, :][None, :], axis=1)
    a2_ref[1, :] = jnp.sum(h2 * a2d_ref[0, :][None, :], axis=1)


def _stage_c(out1, W2p, a2sp, a2dp):
    return pl.pallas_call(
        _stage_c_body,
        grid=(NP // NB,),
        in_specs=[
            pl.BlockSpec((HEADS, NB, HID), lambda i: (0, i, 0)),
            pl.BlockSpec((HEADS * HID, 128), lambda i: (0, 0)),
            pl.BlockSpec((1, 128), lambda i: (0, 0)),
            pl.BlockSpec((1, 128), lambda i: (0, 0)),
        ],
        out_specs=[
            pl.BlockSpec((NB, 16), lambda i: (i, 0)),
            pl.BlockSpec((2, NB), lambda i: (0, i)),
        ],
        out_shape=[
            jax.ShapeDtypeStruct((NP, 16), jnp.float32),
            jax.ShapeDtypeStruct((2, NP), jnp.float32),
        ],
    )(out1, W2p, a2sp, a2dp)


_GROUPS2 = EP // 32 // 512


@functools.partial(
    pl.kernel,
    out_type=jax.ShapeDtypeStruct((2, NP, 16), jnp.float32),
    mesh=_MESH,
    scratch_types=[
        pltpu.VMEM((80, 128), jnp.float32),
        pltpu.VMEM((80, 128), jnp.float32),
        pltpu.VMEM((4, 128), jnp.int32),
        pltpu.VMEM((4, 128), jnp.int32),
        pltpu.VMEM((512, 16), jnp.float32),
        pltpu.VMEM((512,), jnp.float32),
        pltpu.VMEM((16,), jnp.int32),
        pltpu.VMEM_SHARED((NP, 16), jnp.float32),
        pltpu.SemaphoreType.DMA,
        pltpu.SemaphoreType.DMA,
        pltpu.SemaphoreType.DMA,
    ],
    compiler_params=_SCPARAMS,
)
def _l2_edge(h2ext, a22d, pk2d, out2p,
             at1, at2, srci, dsti, rows, wbuf, ibuf, acc, gsem, ssem, isem):
    iota16 = jnp.arange(16, dtype=jnp.int32)
    c = lax.axis_index("c")
    s = lax.axis_index("s")
    wid = s * 2 + c

    pltpu.sync_copy(a22d.at[pl.ds(0, 80)], at1)
    pltpu.sync_copy(a22d.at[pl.ds(80, 80)], at2)

    def zb(i, _):
        rows[i, pl.ds(0, 16)] = jnp.zeros((16,), jnp.float32)
        return 0
    lax.fori_loop(0, 128, zb, 0)
    for z in range(5):
        pltpu.sync_copy(rows.at[pl.ds(0, 128)],
                        acc.at[pl.ds(s * NB + z * 128, 128)])
    plsc.subcore_barrier()

    def group(g, _):
        row0 = wid * (_GROUPS2 * 4) + g * 4
        ibuf[pl.ds(0, 16)] = row0 + iota16
        pltpu.async_copy(pk2d.at[ibuf.at[pl.ds(0, 4)]], srci, isem).wait()

        def unp(i2, _):
            for j in range(4):
                v = srci[j, pl.ds(i2 * 16, 16)]
                dsti[j, pl.ds(i2 * 16, 16)] = v >> 14
                srci[j, pl.ds(i2 * 16, 16)] = v & 16383
            return 0
        lax.fori_loop(0, 8, unp, 0)

        gcps = [
            pltpu.async_copy(h2ext.at[srci.at[j]],
                             rows.at[pl.ds(j * 128, 128)], gsem)
            for j in range(4)
        ]
        for j in range(4):
            def wb(i2, _):
                si = srci[j, pl.ds(i2 * 16, 16)]
                di = dsti[j, pl.ds(i2 * 16, 16)]
                e = (plsc.load_gather(at1, [si >> 7, si & 127])
                     + plsc.load_gather(at2, [di >> 7, di & 127]))
                e = jnp.maximum(e, e * 0.2)
                wbuf[pl.ds(j * 128 + i2 * 16, 16)] = jnp.exp(e)
                return 0
            lax.fori_loop(0, 8, wb, 0)
        for cp in gcps:
            cp.wait()

        def sb(i2, _):
            wv = wbuf[pl.ds(i2 * 16, 16)]
            for k in range(16):
                e2 = i2 * 16 + k
                rows[e2, pl.ds(0, 16)] = rows[e2, pl.ds(0, 16)] * wv[k]
            return 0
        lax.fori_loop(0, 32, sb, 0)

        scps = [
            pltpu.async_copy(rows.at[pl.ds(j * 128, 128)],
                             acc.at[dsti.at[j]], ssem, add=True)
            for j in range(4)
        ]
        for cp in scps:
            cp.wait()
        return 0
    lax.fori_loop(0, _GROUPS2, group, 0)
    plsc.subcore_barrier()
    pltpu.sync_copy(acc.at[pl.ds(s * NB, NB)], out2p.at[c, pl.ds(s * NB, NB)])


def _stage_e_body(p_ref, b2_ref, o_ref):
    sm = p_ref[0] + p_ref[1]
    den = sm[:, 3:4] + 1e-16
    lg = sm / den + b2_ref[0, :][None, :]
    col = lax.broadcasted_iota(jnp.int32, (NB, 16), 1)
    valid = col < ODIM
    m = jnp.max(jnp.where(valid, lg, -1e30), axis=1, keepdims=True)
    ex = jnp.where(valid, jnp.exp(lg - m), 0.0)
    lse = jnp.log(jnp.sum(ex, axis=1, keepdims=True))
    o_ref[...] = lg - m - lse


def _stage_e(out2p, b2p):
    return pl.pallas_call(
        _stage_e_body,
        grid=(NP // NB,),
        in_specs=[
            pl.BlockSpec((2, NB, 16), lambda i: (0, i, 0)),
            pl.BlockSpec((1, 16), lambda i: (0, 0)),
        ],
        out_specs=pl.BlockSpec((NB, 16), lambda i: (i, 0)),
        out_shape=jax.ShapeDtypeStruct((NP, 16), jnp.float32),
    )(out2p, b2p)


def kernel(x, edge_index, W1, a_src1, a_dst1, b1, W2, a_src2, a_dst2, b2):
    ei = edge_index.astype(jnp.int32)
    loop = jnp.arange(N, dtype=jnp.int32)
    padv = jnp.full((EP - E - N,), NP - 1, jnp.int32)
    src = jnp.concatenate([ei[0], loop, padv])
    dst = jnp.concatenate([ei[1], loop, padv])
    pk2d = (src | (dst << 14)).reshape(EROWS, 128)

    xp = jnp.pad(x, ((0, NP - N), (0, 0)))
    W2p = jnp.pad(W2, ((0, 0), (0, 128 - ODIM)))
    a2sp = jnp.pad(a_src2[0], ((0, 0), (0, 128 - ODIM)))
    a2dp = jnp.pad(a_dst2[0], ((0, 0), (0, 128 - ODIM)))
    b2p = jnp.pad(b2, (0, 16 - ODIM)).reshape(1, 16)
    perm = jnp.concatenate([
        jnp.arange(0, 32, 2), jnp.arange(1, 32, 2),
        jnp.arange(32, 64, 2), jnp.arange(33, 64, 2)])
    b1inp = jnp.pad(b1.reshape(HEADS, HID)[:, perm],
                    ((0, 0), (0, 128 - HID)))
    W2p = W2p.reshape(HEADS, HID, 128)[:, perm, :].reshape(HEADS * HID, 128)

    hrows, hb, aso, ado = _stage_a(xp, W1, a_src1, a_dst1)
    del hrows
    hbi = lax.bitcast_convert_type(
        hb.reshape(HEADS, NP, HID // 2, 2), jnp.int32)
    aso2 = aso.reshape(HEADS * 80, 128)
    ado2 = ado.reshape(HEADS * 80, 128)
    w2d, den = _w_kernel(aso2, ado2, pk2d)
    den2d = den.reshape(HEADS * 80, 128)
    out1 = _m_kernel(hbi, pk2d, w2d, den2d, b1inp)
    h2ext, a2 = _stage_c(out1, W2p, a2sp, a2dp)
    a22d = a2.reshape(160, 128)
    out2p = _l2_edge(h2ext, a22d, pk2d)
    res = _stage_e(out2p, b2p)
    return res[:N, :ODIM]
